# baseline (speedup 1.0000x reference)
import jax
import jax.numpy as jnp
from jax.experimental import pallas as pl
from jax.experimental.pallas import tpu as pltpu

N_NODES = 50
N_EDGES = 800
EMB = 5
F32 = jnp.float32


def _lrelu(x):
    return jnp.where(x >= 0, x, 0.01 * x)


def _sigmoid(x):
    return 1.0 / (1.0 + jnp.exp(-x))


def _dot(a, b):
    return jax.lax.dot_general(a, b, (((1,), (0,)), ((), ())),
                               preferred_element_type=F32)


def _fused(ei_ref, dflat_ref, fW1_ref, p_ref, out_ref):
    src = ei_ref[0:1, :]
    dst = ei_ref[1:2, :]
    rows = jax.lax.broadcasted_iota(jnp.int32, (N_NODES, N_EDGES), 0)
    m_dst = (rows == dst).astype(F32)
    m_src = (rows == src).astype(F32)
    A = jax.lax.dot_general(m_dst, m_src, (((1,), (1,)), ((), ())),
                            preferred_element_type=F32)

    x0 = p_ref[344:394, 0:1]
    ax0 = _dot(A, x0)
    t0 = ax0 * p_ref[400:401, 0:EMB]
    h = _lrelu(x0 * p_ref[408:409, 0:EMB] + t0)
    g = _sigmoid(x0 * p_ref[416:417, 0:EMB] + t0)
    x = jnp.concatenate([h, g * h], axis=1)

    for base in (424, 472):
        W = p_ref[base:base + 10, 0:EMB]
        U = p_ref[base + 16:base + 26, 0:EMB]
        G = p_ref[base + 32:base + 42, 0:EMB]
        ax = _dot(A, x)
        t = _dot(ax, W)
        h = _lrelu(_dot(x, U) + t)
        g = _sigmoid(_dot(x, G) + t)
        x = jnp.concatenate([h, g * h], axis=1)

    xt = jnp.transpose(x)
    vecx = jnp.concatenate([xt[j:j + 1, :] for j in range(2 * EMB)], axis=1)
    nvec = N_NODES * 2 * EMB
    pr = jax.lax.broadcasted_iota(jnp.int32, (nvec, nvec), 0)
    pc = jax.lax.broadcasted_iota(jnp.int32, (nvec, nvec), 1)
    perm = (pc == (pr % N_NODES) * (2 * EMB) + pr // N_NODES).astype(F32)
    vecx = _dot(vecx, perm)
    full = jnp.concatenate([vecx, dflat_ref[...]], axis=1)

    fW2 = p_ref[0:128, :]
    fb1 = p_ref[128:129, :]
    fb2 = p_ref[129:130, :]
    fW3 = p_ref[136:264, 0:64]
    fb3 = p_ref[328:329, 0:64]
    fW4 = p_ref[264:328, 0:2]
    fb4 = p_ref[336:337, 0:2]
    h1 = _lrelu(_dot(full, fW1_ref[...]) + fb1)
    h2 = _lrelu(_dot(h1, fW2) + fb2)
    h3 = _lrelu(_dot(h2, fW3) + fb3)
    y = _sigmoid(_dot(h3, fW4) + fb4)
    out_ref[...] = 0.5 * (y[0, 0] + y[0, 1])


def kernel(data, d, edge_index, W0, U0, G0, W1, U1, G1, W2, U2, G2,
           fW1, fb1, fW2, fb2, fW3, fb3, fW4, fb4):
    dflat = d.reshape(1, -1)

    def pad(a, rows):
        return jnp.pad(a, ((0, rows - a.shape[0]), (0, 128 - a.shape[1])))

    packed = jnp.concatenate([
        fW2,
        fb1.reshape(1, -1), fb2.reshape(1, -1),
        jnp.zeros((6, 128), F32),
        pad(fW3, 128),
        pad(fW4, 64),
        pad(fb3.reshape(1, -1), 8),
        pad(fb4.reshape(1, -1), 8),
        pad(data, 56),
        pad(W0, 8), pad(U0, 8), pad(G0, 8),
        pad(W1, 16), pad(U1, 16), pad(G1, 16),
        pad(W2, 16), pad(U2, 16), pad(G2, 16),
    ], axis=0)
    out = pl.pallas_call(
        _fused,
        out_shape=jax.ShapeDtypeStruct((), F32),
        out_specs=pl.BlockSpec(memory_space=pltpu.SMEM),
        compiler_params=pltpu.CompilerParams(
            allow_input_fusion=(True,) * 4),
    )(edge_index, dflat, fW1, packed)
    return out
